# baseline (speedup 1.0000x reference)
import functools
import jax
import jax.numpy as jnp
from jax import lax
from jax.experimental import pallas as pl
from jax.experimental.pallas import tpu as pltpu
from jax.experimental.pallas import tpu_sc as plsc

_M = 100000
_D = 64
_B = 1024
_BB = 32
_NB = _B // _BB
_NC = 2
_NS = 16
_NW = _NC * _NS
_BPW = _B // _NW


@functools.partial(
    pl.kernel,
    out_type=jax.ShapeDtypeStruct((_B, _D), jnp.float32),
    mesh=plsc.VectorSubcoreMesh(core_axis_name="c", subcore_axis_name="s"),
    compiler_params=pltpu.CompilerParams(use_tc_tiling_on_sc=False),
    scratch_types=[
        pltpu.VMEM((_BPW,), jnp.int32),
        pltpu.VMEM((_BPW, _D), jnp.float32),
        pltpu.SemaphoreType.DMA,
    ],
)
def _sc_gather(feat_hbm, idx_hbm, out_hbm, idx_v, rows_v, sem):
    wid = lax.axis_index("s") * _NC + lax.axis_index("c")
    base = wid * _BPW
    pltpu.sync_copy(idx_hbm.at[pl.ds(base, _BPW)], idx_v)
    pltpu.async_copy(feat_hbm.at[idx_v], rows_v, sem).wait()
    pltpu.sync_copy(rows_v, out_hbm.at[pl.ds(base, _BPW)])


def _tc_prep_body(m_ref, x_ref, g_ref, xnb_ref, upd_ref):
    x = x_ref[...]
    xn = x / (jnp.sqrt(jnp.sum(x * x, axis=1, keepdims=True)) + 1e-12)
    xnb_ref[...] = xn.astype(jnp.bfloat16)
    m = m_ref[0, 0]
    upd = m * g_ref[...] + (1.0 - m) * xn
    upd_ref[...] = upd / (
        jnp.sqrt(jnp.sum(upd * upd, axis=1, keepdims=True)) + 1e-12)


def _tc_slab_body(xnb_ref, featb_ref, out_ref, sb0, sb1, sems):
    i = pl.program_id(0)
    slot = lax.rem(i, 2)
    sbs = [sb0, sb1]
    for k in range(2):
        @pl.when((slot == k) & (i >= 2))
        def _drain(k=k):
            pltpu.make_async_copy(
                sbs[k], out_ref.at[pl.ds((i - 2) * _BB, _BB), :],
                sems.at[k]).wait()

        @pl.when(slot == k)
        def _compute_and_issue(k=k):
            sbs[k][...] = lax.dot_general(
                xnb_ref[...], featb_ref[...],
                (((1,), (1,)), ((), ())), preferred_element_type=jnp.float32)
            pltpu.make_async_copy(
                sbs[k], out_ref.at[pl.ds(i * _BB, _BB), :],
                sems.at[k]).start()

    @pl.when(i == _NB - 1)
    def _drain_all():
        for k in range(2):
            j = _NB - 1 - k
            pltpu.make_async_copy(
                sbs[j % 2], out_ref.at[pl.ds(j * _BB, _BB), :],
                sems.at[j % 2]).wait()


def kernel(inputs, indexes, features, momentum):
    g = _sc_gather(features, indexes)

    m2 = jnp.asarray(momentum, jnp.float32).reshape(1, 1)
    xnb, upd = pl.pallas_call(
        _tc_prep_body,
        in_specs=[
            pl.BlockSpec(memory_space=pltpu.SMEM),
            pl.BlockSpec((_B, _D), lambda: (0, 0)),
            pl.BlockSpec((_B, _D), lambda: (0, 0)),
        ],
        out_specs=[
            pl.BlockSpec((_B, _D), lambda: (0, 0)),
            pl.BlockSpec((_B, _D), lambda: (0, 0)),
        ],
        out_shape=[
            jax.ShapeDtypeStruct((_B, _D), jnp.bfloat16),
            jax.ShapeDtypeStruct((_B, _D), jnp.float32),
        ],
    )(m2, inputs, g)

    featb = features.astype(jnp.bfloat16)
    out = pl.pallas_call(
        _tc_slab_body,
        grid=(_NB,),
        in_specs=[
            pl.BlockSpec((_BB, _D), lambda i: (i, 0)),
            pl.BlockSpec((_M, _D), lambda i: (0, 0)),
        ],
        out_specs=pl.BlockSpec(memory_space=pltpu.HBM),
        out_shape=jax.ShapeDtypeStruct((_B, _M), jnp.float32),
        scratch_shapes=[
            pltpu.VMEM((_BB, _M), jnp.float32),
            pltpu.VMEM((_BB, _M), jnp.float32),
            pltpu.SemaphoreType.DMA((2,)),
        ],
    )(xnb, featb)
    return out, features


# speedup vs baseline: 1.1898x; 1.1898x over previous
import functools
import jax
import jax.numpy as jnp
from jax import lax
from jax.experimental import pallas as pl
from jax.experimental.pallas import tpu as pltpu
from jax.experimental.pallas import tpu_sc as plsc

_M = 100000
_D = 64
_B = 1024
_TM = 1280
_GRID = (_M + _TM - 1) // _TM
_LAST = _M - (_GRID - 1) * _TM
_NC = 2
_NS = 16
_NW = _NC * _NS
_BPW = _B // _NW


@functools.partial(
    pl.kernel,
    out_type=jax.ShapeDtypeStruct((_B, _D), jnp.float32),
    mesh=plsc.VectorSubcoreMesh(core_axis_name="c", subcore_axis_name="s"),
    compiler_params=pltpu.CompilerParams(use_tc_tiling_on_sc=False),
    scratch_types=[
        pltpu.VMEM((_BPW,), jnp.int32),
        pltpu.VMEM((_BPW, _D), jnp.float32),
        pltpu.SemaphoreType.DMA,
    ],
)
def _sc_gather(feat_hbm, idx_hbm, out_hbm, idx_v, rows_v, sem):
    wid = lax.axis_index("s") * _NC + lax.axis_index("c")
    base = wid * _BPW
    pltpu.sync_copy(idx_hbm.at[pl.ds(base, _BPW)], idx_v)
    pltpu.async_copy(feat_hbm.at[idx_v], rows_v, sem).wait()
    pltpu.sync_copy(rows_v, out_hbm.at[pl.ds(base, _BPW)])


def _tc_prep_body(m_ref, x_ref, g_ref, xnb_ref, upd_ref):
    x = x_ref[...]
    xn = x / (jnp.sqrt(jnp.sum(x * x, axis=1, keepdims=True)) + 1e-12)
    xnb_ref[...] = xn.astype(jnp.bfloat16)
    m = m_ref[0, 0]
    upd = m * g_ref[...] + (1.0 - m) * xn
    upd_ref[...] = upd / (
        jnp.sqrt(jnp.sum(upd * upd, axis=1, keepdims=True)) + 1e-12)


def _tc_mm_body(idx_ref, xnb_ref, upd_ref, feat_ref, out_ref, newf_ref):
    i = pl.program_id(0)

    feat = feat_ref[...]
    out_ref[...] = lax.dot_general(
        xnb_ref[...], feat.astype(jnp.bfloat16),
        (((1,), (1,)), ((), ())), preferred_element_type=jnp.float32)

    @pl.when(i < _GRID - 1)
    def _copy_full():
        newf_ref[pl.ds(i * _TM, _TM), :] = feat

    @pl.when(i == _GRID - 1)
    def _copy_tail():
        newf_ref[pl.ds((_GRID - 1) * _TM, _LAST), :] = feat[:_LAST, :]

    def body(b, carry):
        newf_ref[pl.ds(idx_ref[b], 1), :] = upd_ref[pl.ds(b, 1), :]
        return carry

    lax.fori_loop(jnp.where(i == _GRID - 1, 0, _B), _B, body, 0)


def kernel(inputs, indexes, features, momentum):
    g = _sc_gather(features, indexes)

    m2 = jnp.asarray(momentum, jnp.float32).reshape(1, 1)
    xnb, upd = pl.pallas_call(
        _tc_prep_body,
        in_specs=[
            pl.BlockSpec(memory_space=pltpu.SMEM),
            pl.BlockSpec((_B, _D), lambda: (0, 0)),
            pl.BlockSpec((_B, _D), lambda: (0, 0)),
        ],
        out_specs=[
            pl.BlockSpec((_B, _D), lambda: (0, 0)),
            pl.BlockSpec((_B, _D), lambda: (0, 0)),
        ],
        out_shape=[
            jax.ShapeDtypeStruct((_B, _D), jnp.bfloat16),
            jax.ShapeDtypeStruct((_B, _D), jnp.float32),
        ],
    )(m2, inputs, g)

    out, newf = pl.pallas_call(
        _tc_mm_body,
        grid=(_GRID,),
        compiler_params=pltpu.CompilerParams(vmem_limit_bytes=100 * 2**20),
        in_specs=[
            pl.BlockSpec(memory_space=pltpu.SMEM),
            pl.BlockSpec((_B, _D), lambda i: (0, 0)),
            pl.BlockSpec((_B, _D), lambda i: (0, 0)),
            pl.BlockSpec((_TM, _D), lambda i: (i, 0)),
        ],
        out_specs=[
            pl.BlockSpec((_B, _TM), lambda i: (0, i)),
            pl.BlockSpec((_M, _D), lambda i: (0, 0)),
        ],
        out_shape=[
            jax.ShapeDtypeStruct((_B, _M), jnp.float32),
            jax.ShapeDtypeStruct((_M, _D), jnp.float32),
        ],
    )(indexes, xnb, upd, features)
    return out, newf
